# baseline (speedup 1.0000x reference)
import functools

import jax
import jax.numpy as jnp
from jax import lax
from jax.experimental import pallas as pl
from jax.experimental.pallas import tpu as pltpu
from jax.experimental.pallas import tpu_sc as plsc

B = 16384
EMB = 64
NUM_CORES = 2
NUM_SUBCORES = 16
NW = NUM_CORES * NUM_SUBCORES
BPW = B // NW

_SC_PARAMS = pltpu.CompilerParams(use_tc_tiling_on_sc=False)


def _sc_gather_pair(uid, iid, U, I, fuse_mul):
    mesh = plsc.VectorSubcoreMesh(core_axis_name="c", subcore_axis_name="s")
    n_out = 1 if fuse_mul else 2

    @functools.partial(
        pl.kernel,
        mesh=mesh,
        compiler_params=_SC_PARAMS,
        out_type=[jax.ShapeDtypeStruct((B, EMB), jnp.float32)] * n_out,
        scratch_types=[
            pltpu.VMEM((BPW,), jnp.int32),
            pltpu.VMEM((BPW,), jnp.int32),
            pltpu.VMEM((BPW, EMB), jnp.float32),
            pltpu.VMEM((BPW, EMB), jnp.float32),
            pltpu.SemaphoreType.DMA,
            pltpu.SemaphoreType.DMA,
        ],
    )
    def k(uid_hbm, iid_hbm, u_hbm, i_hbm, *out_and_scratch):
        outs = out_and_scratch[:n_out]
        idx_u, idx_i, buf_a, buf_b, sem_a, sem_b = out_and_scratch[n_out:]
        wid = lax.axis_index("s") * NUM_CORES + lax.axis_index("c")
        base = wid * BPW
        pltpu.sync_copy(uid_hbm.at[pl.ds(base, BPW)], idx_u)
        pltpu.sync_copy(iid_hbm.at[pl.ds(base, BPW)], idx_i)
        cp_a = pltpu.async_copy(u_hbm.at[idx_u], buf_a, sem_a)
        cp_b = pltpu.async_copy(i_hbm.at[idx_i], buf_b, sem_b)
        cp_a.wait()
        cp_b.wait()
        if fuse_mul:
            def mul_row(i, carry):
                for j in range(EMB // 16):
                    sl = pl.ds(j * 16, 16)
                    buf_a[i, sl] = buf_a[i, sl] * buf_b[i, sl]
                return carry

            lax.fori_loop(0, BPW, mul_row, 0)
            pltpu.sync_copy(buf_a, outs[0].at[pl.ds(base, BPW)])
        else:
            pltpu.sync_copy(buf_a, outs[0].at[pl.ds(base, BPW)])
            pltpu.sync_copy(buf_b, outs[1].at[pl.ds(base, BPW)])

    return k(uid, iid, U, I)


def _tc_mlp(um, im, W1, b1, W2, b2, W3, b3):
    w1u = W1[:, :EMB].T
    w1i = W1[:, EMB:].T
    w2t = W2.T
    w3t = W3.T
    b1r = b1.reshape(1, -1)
    b2r = b2.reshape(1, -1)
    b3r = b3.reshape(1, -1)

    BLK = 2048
    h0 = W1.shape[0]
    h1 = W2.shape[0]
    h2 = W3.shape[0]

    def body(um_ref, im_ref, w1u_ref, w1i_ref, b1_ref, w2_ref, b2_ref,
             w3_ref, b3_ref, out_ref):
        x = jnp.dot(um_ref[...], w1u_ref[...], preferred_element_type=jnp.float32)
        x = x + jnp.dot(im_ref[...], w1i_ref[...], preferred_element_type=jnp.float32)
        x = jnp.maximum(x + b1_ref[...], 0.0)
        x = jnp.dot(x, w2_ref[...], preferred_element_type=jnp.float32)
        x = jnp.maximum(x + b2_ref[...], 0.0)
        x = jnp.dot(x, w3_ref[...], preferred_element_type=jnp.float32)
        out_ref[...] = jnp.maximum(x + b3_ref[...], 0.0)

    full = lambda r, c: pl.BlockSpec((r, c), lambda i: (0, 0))
    return pl.pallas_call(
        body,
        grid=(B // BLK,),
        in_specs=[
            pl.BlockSpec((BLK, EMB), lambda i: (i, 0)),
            pl.BlockSpec((BLK, EMB), lambda i: (i, 0)),
            full(EMB, h0),
            full(EMB, h0),
            full(1, h0),
            full(h0, h1),
            full(1, h1),
            full(h1, h2),
            full(1, h2),
        ],
        out_specs=pl.BlockSpec((BLK, h2), lambda i: (i, 0)),
        out_shape=jax.ShapeDtypeStruct((B, h2), jnp.float32),
    )(um, im, w1u, w1i, b1r, w2t, b2r, w3t, b3r)


def _tc_final(gmf, h, Wp, bp):
    wpg = Wp[:, :EMB]
    wpx = Wp[:, EMB:]
    bpr = jnp.reshape(bp, (1, 1))
    BLK = 4096
    h2 = h.shape[1]

    def body(g_ref, h_ref, wpg_ref, wpx_ref, bp_ref, out_ref):
        logit = (jnp.sum(g_ref[...] * wpg_ref[...], axis=1, keepdims=True)
                 + jnp.sum(h_ref[...] * wpx_ref[...], axis=1, keepdims=True)
                 + bp_ref[0, 0])
        out_ref[...] = 1.0 / (1.0 + jnp.exp(-logit))

    full = lambda r, c: pl.BlockSpec((r, c), lambda i: (0, 0))
    out = pl.pallas_call(
        body,
        grid=(B // BLK,),
        in_specs=[
            pl.BlockSpec((BLK, EMB), lambda i: (i, 0)),
            pl.BlockSpec((BLK, h2), lambda i: (i, 0)),
            full(1, EMB),
            full(1, h2),
            full(1, 1),
        ],
        out_specs=pl.BlockSpec((BLK, 1), lambda i: (i, 0)),
        out_shape=jax.ShapeDtypeStruct((B, 1), jnp.float32),
    )(gmf, h, wpg, wpx, bpr)
    return jnp.squeeze(out, axis=-1)


def kernel(user_ids, item_ids, Ug, Ig, Um, Im, W1, b1, W2, b2, W3, b3, Wp, bp):
    uid = user_ids.astype(jnp.int32)
    iid = item_ids.astype(jnp.int32)
    um, im = _sc_gather_pair(uid, iid, Um, Im, fuse_mul=False)
    (gmf,) = _sc_gather_pair(uid, iid, Ug, Ig, fuse_mul=True)
    h = _tc_mlp(um, im, W1, b1, W2, b2, W3, b3)
    return _tc_final(gmf, h, Wp, bp)


# speedup vs baseline: 1.1007x; 1.1007x over previous
import functools

import jax
import jax.numpy as jnp
from jax import lax
from jax.experimental import pallas as pl
from jax.experimental.pallas import tpu as pltpu
from jax.experimental.pallas import tpu_sc as plsc

B = 16384
EMB = 64
NUM_CORES = 2
NUM_SUBCORES = 16
NW = NUM_CORES * NUM_SUBCORES
BPW = B // NW

_SC_PARAMS = pltpu.CompilerParams(use_tc_tiling_on_sc=False)


def _sc_gather_gmf(uid, iid, Ug, Ig, Um, Im):
    mesh = plsc.VectorSubcoreMesh(core_axis_name="c", subcore_axis_name="s")

    @functools.partial(
        pl.kernel,
        mesh=mesh,
        compiler_params=_SC_PARAMS,
        out_type=[
            jax.ShapeDtypeStruct((B, 2 * EMB), jnp.float32),
            jax.ShapeDtypeStruct((B, 2 * EMB), jnp.float32),
        ],
        scratch_types=[
            pltpu.VMEM((BPW,), jnp.int32),
            pltpu.VMEM((BPW,), jnp.int32),
            pltpu.VMEM((BPW, EMB), jnp.float32),
            pltpu.VMEM((BPW, EMB), jnp.float32),
            pltpu.SemaphoreType.DMA,
            pltpu.SemaphoreType.DMA,
        ],
    )
    def k(uid_hbm, iid_hbm, ug_hbm, ig_hbm, um_hbm, im_hbm,
          x_out, g_out, idx_u, idx_i, buf_a, buf_b, sem_a, sem_b):
        wid = lax.axis_index("s") * NUM_CORES + lax.axis_index("c")
        base = wid * BPW
        rows = pl.ds(base, BPW)
        pltpu.sync_copy(uid_hbm.at[rows], idx_u)
        pltpu.sync_copy(iid_hbm.at[rows], idx_i)
        cp_a = pltpu.async_copy(um_hbm.at[idx_u], buf_a, sem_a)
        cp_b = pltpu.async_copy(im_hbm.at[idx_i], buf_b, sem_b)
        cp_a.wait()
        cp_b.wait()
        pltpu.sync_copy(buf_a, x_out.at[rows, pl.ds(0, EMB)])
        pltpu.sync_copy(buf_b, x_out.at[rows, pl.ds(EMB, EMB)])
        cp_a = pltpu.async_copy(ug_hbm.at[idx_u], buf_a, sem_a)
        cp_b = pltpu.async_copy(ig_hbm.at[idx_i], buf_b, sem_b)
        cp_a.wait()
        cp_b.wait()

        def mul_row(i, carry):
            for j in range(EMB // 16):
                sl = pl.ds(j * 16, 16)
                buf_a[i, sl] = buf_a[i, sl] * buf_b[i, sl]
            return carry

        lax.fori_loop(0, BPW, mul_row, 0)
        pltpu.sync_copy(buf_a, g_out.at[rows, pl.ds(0, EMB)])

    return k(uid, iid, Ug, Ig, Um, Im)


def _tc_mlp_final(x_in, g, W1, b1, W2, b2, W3, b3, Wp, bp):
    w1t = W1.T
    w2t = W2.T
    w3t = W3.T
    wpg = jnp.concatenate(
        [Wp[:, :EMB], jnp.zeros((1, EMB), jnp.float32)], axis=1)
    wpx = Wp[:, EMB:]
    b1r = b1.reshape(1, -1)
    b2r = b2.reshape(1, -1)
    b3r = b3.reshape(1, -1)
    bpr = jnp.reshape(bp, (1, 1))

    BLK = 2048
    h0 = W1.shape[0]
    h1 = W2.shape[0]
    h2 = W3.shape[0]

    def body(x_ref, g_ref, w1_ref, b1_ref, w2_ref, b2_ref, w3_ref, b3_ref,
             wpg_ref, wpx_ref, bp_ref, out_ref):
        x = jnp.dot(x_ref[...], w1_ref[...], preferred_element_type=jnp.float32)
        x = jnp.maximum(x + b1_ref[...], 0.0)
        x = jnp.dot(x, w2_ref[...], preferred_element_type=jnp.float32)
        x = jnp.maximum(x + b2_ref[...], 0.0)
        x = jnp.dot(x, w3_ref[...], preferred_element_type=jnp.float32)
        x = jnp.maximum(x + b3_ref[...], 0.0)
        logit = (jnp.sum(g_ref[...] * wpg_ref[...], axis=1, keepdims=True)
                 + jnp.sum(x * wpx_ref[...], axis=1, keepdims=True)
                 + bp_ref[0, 0])
        out_ref[...] = 1.0 / (1.0 + jnp.exp(-logit))

    full = lambda r, c: pl.BlockSpec((r, c), lambda i: (0, 0))
    out = pl.pallas_call(
        body,
        grid=(B // BLK,),
        in_specs=[
            pl.BlockSpec((BLK, 2 * EMB), lambda i: (i, 0)),
            pl.BlockSpec((BLK, 2 * EMB), lambda i: (i, 0)),
            full(2 * EMB, h0),
            full(1, h0),
            full(h0, h1),
            full(1, h1),
            full(h1, h2),
            full(1, h2),
            full(1, 2 * EMB),
            full(1, h2),
            full(1, 1),
        ],
        out_specs=pl.BlockSpec((BLK, 1), lambda i: (i, 0)),
        out_shape=jax.ShapeDtypeStruct((B, 1), jnp.float32),
    )(x_in, g, w1t, b1r, w2t, b2r, w3t, b3r, wpg, wpx, bpr)
    return jnp.squeeze(out, axis=-1)


def kernel(user_ids, item_ids, Ug, Ig, Um, Im, W1, b1, W2, b2, W3, b3, Wp, bp):
    uid = user_ids.astype(jnp.int32)
    iid = item_ids.astype(jnp.int32)
    x_in, g = _sc_gather_gmf(uid, iid, Ug, Ig, Um, Im)
    return _tc_mlp_final(x_in, g, W1, b1, W2, b2, W3, b3, Wp, bp)
